# baseline (speedup 1.0000x reference)
import functools

import jax
import jax.numpy as jnp
from jax import lax
from jax.experimental import pallas as pl
from jax.experimental.pallas import tpu as pltpu
from jax.experimental.pallas import tpu_sc as plsc

N = 10000
NP = 10240
E = 320000
H = 128
G = 256
T = 10

NC = 2
NS = 16
EPT = E // (NC * NS)
CH = 125
NCH = EPT // CH
GRP = 40
NG = NCH // GRP
RPT = NP // NS


def _make_agg(nh: int):
    mesh = plsc.VectorSubcoreMesh(core_axis_name="c", subcore_axis_name="s")
    out_type = jax.ShapeDtypeStruct((NC, NP, H), jnp.float32)
    scratch = [
        pltpu.VMEM_SHARED((NP, H), jnp.float32),
        pltpu.VMEM((GRP, CH), jnp.int32),
        pltpu.VMEM((GRP, CH), jnp.int32),
        pltpu.VMEM((CH, H), jnp.float32),
        pltpu.VMEM((CH, H), jnp.float32),
        pltpu.SemaphoreType.DMA,
        pltpu.SemaphoreType.DMA,
        pltpu.SemaphoreType.DMA,
        pltpu.SemaphoreType.DMA,
    ]

    def body(h_hbm, src_hbm, dst_hbm, z2_hbm, agg_hbm,
             agg_sh, srcv, dstv, rows0, rows1, sem0, sem1, ses0, ses1):
        c = lax.axis_index("c")
        s = lax.axis_index("s")
        w = c * NS + s
        pltpu.sync_copy(z2_hbm, agg_sh.at[pl.ds(s * RPT, RPT)])
        plsc.subcore_barrier()

        def group(g, carry):
            r0 = w * NCH + g * GRP
            pltpu.sync_copy(src_hbm.at[pl.ds(r0, GRP)], srcv)
            pltpu.sync_copy(dst_hbm.at[pl.ds(r0, GRP)], dstv)
            pltpu.async_copy(h_hbm.at[srcv.at[0]], rows0, sem0)
            pltpu.async_copy(h_hbm.at[srcv.at[1]], rows1, sem1)

            def it(p, carry2):
                i = 2 * p
                pltpu.make_async_copy(h_hbm.at[srcv.at[i]], rows0, sem0).wait()
                pltpu.sync_copy(rows0, agg_sh.at[dstv.at[i]], add=True)
                pltpu.async_copy(h_hbm.at[srcv.at[i + 2]], rows0, sem0)
                pltpu.make_async_copy(h_hbm.at[srcv.at[i + 1]], rows1,
                                      sem1).wait()
                pltpu.sync_copy(rows1, agg_sh.at[dstv.at[i + 1]], add=True)
                pltpu.async_copy(h_hbm.at[srcv.at[i + 3]], rows1, sem1)
                return carry2

            lax.fori_loop(0, GRP // 2 - 1, it, 0)
            pltpu.make_async_copy(h_hbm.at[srcv.at[GRP - 2]], rows0,
                                  sem0).wait()
            pltpu.sync_copy(rows0, agg_sh.at[dstv.at[GRP - 2]], add=True)
            pltpu.make_async_copy(h_hbm.at[srcv.at[GRP - 1]], rows1,
                                  sem1).wait()
            pltpu.sync_copy(rows1, agg_sh.at[dstv.at[GRP - 1]], add=True)
            return carry

        lax.fori_loop(0, NG, group, 0)
        plsc.subcore_barrier()
        pltpu.sync_copy(agg_sh.at[pl.ds(s * RPT, RPT)],
                        agg_hbm.at[c, pl.ds(s * RPT, RPT)])

    return pl.kernel(body, mesh=mesh, out_type=out_type, scratch_types=scratch)


def _scatter_agg(h, src2d, dst2d):
    z2 = jnp.zeros((RPT, H), jnp.float32)
    return _make_agg(h.shape[0])(h, src2d, dst2d, z2)


def _make_count():
    mesh = plsc.VectorSubcoreMesh(core_axis_name="c", subcore_axis_name="s")
    out_type = jax.ShapeDtypeStruct((NC, NP, H), jnp.float32)
    KD = 8
    scratch = [
        pltpu.VMEM_SHARED((NP, H), jnp.float32),
        pltpu.VMEM((GRP, CH), jnp.int32),
        pltpu.VMEM((CH, H), jnp.float32),
        pltpu.SemaphoreType.DMA,
    ]

    def body(dst_hbm, z2_hbm, o2_hbm, cnt_hbm, cnt_sh, dstv, onesv, sem):
        c = lax.axis_index("c")
        s = lax.axis_index("s")
        w = c * NS + s
        pltpu.sync_copy(z2_hbm, cnt_sh.at[pl.ds(s * RPT, RPT)])
        pltpu.sync_copy(o2_hbm, onesv)
        plsc.subcore_barrier()

        def group(g, carry):
            pltpu.sync_copy(dst_hbm.at[pl.ds(w * NCH + g * GRP, GRP)], dstv)

            def it(q, carry2):
                for k in range(KD):
                    pltpu.async_copy(onesv, cnt_sh.at[dstv.at[q * KD + k]],
                                     sem, add=True)
                for k in range(KD):
                    pltpu.make_async_copy(onesv,
                                          cnt_sh.at[dstv.at[q * KD + k]],
                                          sem).wait()
                return carry2

            lax.fori_loop(0, GRP // KD, it, 0)
            return carry

        lax.fori_loop(0, NG, group, 0)
        plsc.subcore_barrier()
        pltpu.sync_copy(cnt_sh.at[pl.ds(s * RPT, RPT)],
                        cnt_hbm.at[c, pl.ds(s * RPT, RPT)])

    return pl.kernel(body, mesh=mesh, out_type=out_type, scratch_types=scratch)


def _count(dst2d):
    z2 = jnp.zeros((RPT, H), jnp.float32)
    o2 = jnp.ones((CH, H), jnp.float32)
    return _make_count()(dst2d, z2, o2)


def _mlp0_body(x_ref, wa_ref, wb_ref, o_ref):
    h = jnp.maximum(
        jnp.dot(x_ref[...], wa_ref[...], preferred_element_type=jnp.float32), 0.0)
    o_ref[...] = jnp.maximum(
        jnp.dot(h, wb_ref[...], preferred_element_type=jnp.float32), 0.0)


def _mlp0(x, wa, wb):
    n = x.shape[0]
    blk = 1000
    return pl.pallas_call(
        _mlp0_body,
        grid=(n // blk,),
        in_specs=[pl.BlockSpec((blk, H), lambda i: (i, 0)),
                  pl.BlockSpec((H, H), lambda i: (0, 0)),
                  pl.BlockSpec((H, H), lambda i: (0, 0))],
        out_specs=pl.BlockSpec((blk, H), lambda i: (i, 0)),
        out_shape=jax.ShapeDtypeStruct((n, H), jnp.float32),
    )(x, wa, wb)


def _mlp1_body(a0_ref, a1_ref, c0_ref, c1_ref, wa_ref, wb_ref,
               o_ref, inv_ref):
    cnt = c0_ref[:, 0] + c1_ref[:, 0]
    inv = 1.0 / jnp.maximum(cnt, 1.0)
    inv_ref[...] = inv[:, None]
    mean = (a0_ref[...] + a1_ref[...]) * inv[:, None]
    h = jnp.maximum(
        jnp.dot(mean, wa_ref[...], preferred_element_type=jnp.float32), 0.0)
    o_ref[...] = jnp.maximum(
        jnp.dot(h, wb_ref[...], preferred_element_type=jnp.float32), 0.0)


def _mlp1(a0, a1, c0, c1, wa, wb):
    blk = 1280
    return pl.pallas_call(
        _mlp1_body,
        grid=(NP // blk,),
        in_specs=[pl.BlockSpec((blk, H), lambda i: (i, 0)),
                  pl.BlockSpec((blk, H), lambda i: (i, 0)),
                  pl.BlockSpec((blk, H), lambda i: (i, 0)),
                  pl.BlockSpec((blk, H), lambda i: (i, 0)),
                  pl.BlockSpec((H, H), lambda i: (0, 0)),
                  pl.BlockSpec((H, H), lambda i: (0, 0))],
        out_specs=[pl.BlockSpec((blk, H), lambda i: (i, 0)),
                   pl.BlockSpec((blk, 1), lambda i: (i, 0))],
        out_shape=[jax.ShapeDtypeStruct((NP, H), jnp.float32),
                   jax.ShapeDtypeStruct((NP, 1), jnp.float32)],
    )(a0, a1, c0, c1, wa, wb)


def _mlp2_head_body(a0_ref, a1_ref, inv_ref, wa_ref, wb_ref, b_ref,
                    wout_ref, bout_ref, o_ref, p_ref, psum, pcnt):
    i = pl.program_id(0)
    nb = pl.num_programs(0)

    @pl.when(i == 0)
    def _():
        psum[...] = jnp.zeros_like(psum)
        pcnt[...] = jnp.zeros_like(pcnt)

    mean = (a0_ref[...] + a1_ref[...]) * inv_ref[...]
    h = jnp.maximum(
        jnp.dot(mean, wa_ref[...], preferred_element_type=jnp.float32), 0.0)
    x3 = jnp.maximum(
        jnp.dot(h, wb_ref[...], preferred_element_type=jnp.float32), 0.0)
    o_ref[...] = x3

    b = b_ref[0, 0, :]
    blk = b.shape[0]
    gids = lax.broadcasted_iota(jnp.int32, (blk, G), 1)
    mask = (b[:, None] == gids).astype(jnp.float32)
    psum[...] += lax.dot_general(mask, x3,
                                 (((0,), (0,)), ((), ())),
                                 preferred_element_type=jnp.float32)
    pcnt[...] += lax.dot_general(mask, jnp.ones((blk, H), jnp.float32),
                                 (((0,), (0,)), ((), ())),
                                 preferred_element_type=jnp.float32)

    @pl.when(i == nb - 1)
    def _():
        pooled = psum[...] / jnp.maximum(pcnt[...], 1.0)
        logits = jnp.dot(pooled, wout_ref[...],
                         preferred_element_type=jnp.float32) + bout_ref[...]
        m = jnp.max(logits, axis=-1, keepdims=True)
        e = jnp.exp(logits - m)
        p = e / jnp.sum(e, axis=-1, keepdims=True)
        p_ref[...] = jnp.clip(p, 1e-8, 1.0)


def _mlp2_head(a0, a1, inv, wa, wb, batch3d, wout, bout2d):
    blk = 1280
    return pl.pallas_call(
        _mlp2_head_body,
        grid=(NP // blk,),
        in_specs=[pl.BlockSpec((blk, H), lambda i: (i, 0)),
                  pl.BlockSpec((blk, H), lambda i: (i, 0)),
                  pl.BlockSpec((blk, 1), lambda i: (i, 0)),
                  pl.BlockSpec((H, H), lambda i: (0, 0)),
                  pl.BlockSpec((H, H), lambda i: (0, 0)),
                  pl.BlockSpec((1, 1, blk), lambda i: (i, 0, 0)),
                  pl.BlockSpec((H, T), lambda i: (0, 0)),
                  pl.BlockSpec((1, T), lambda i: (0, 0))],
        out_specs=[pl.BlockSpec((blk, H), lambda i: (i, 0)),
                   pl.BlockSpec((G, T), lambda i: (0, 0))],
        out_shape=[jax.ShapeDtypeStruct((NP, H), jnp.float32),
                   jax.ShapeDtypeStruct((G, T), jnp.float32)],
        scratch_shapes=[pltpu.VMEM((G, H), jnp.float32),
                        pltpu.VMEM((G, H), jnp.float32)],
    )(a0, a1, inv, wa, wb, batch3d, wout, bout2d)


def kernel(x, edge_index, edge_attr, batch, W0a, W0b, W1a, W1b, W2a, W2b,
           Wout, bout):
    src2d = edge_index[0].reshape(E // CH, CH)
    dst2d = edge_index[1].reshape(E // CH, CH)

    cnt = _count(dst2d)
    x1 = _mlp0(x, W0a, W0b)
    agg1 = _scatter_agg(x1, src2d, dst2d)
    x2p, inv = _mlp1(agg1[0], agg1[1], cnt[0], cnt[1], W1a, W1b)
    agg2 = _scatter_agg(x2p, src2d, dst2d)

    batch_pad = jnp.concatenate(
        [batch, jnp.full((NP - N,), G, jnp.int32)]).reshape(NP // 1280, 1, 1280)
    x3p, p = _mlp2_head(agg2[0], agg2[1], inv, W2a, W2b, batch_pad,
                        Wout, bout.reshape(1, T))

    node_embeddings = jnp.concatenate([x1, x2p[:N], x3p[:N]], axis=1)
    return (p, node_embeddings)


# speedup vs baseline: 1.0535x; 1.0535x over previous
import functools

import jax
import jax.numpy as jnp
from jax import lax
from jax.experimental import pallas as pl
from jax.experimental.pallas import tpu as pltpu
from jax.experimental.pallas import tpu_sc as plsc

N = 10000
NP = 10240
E = 320000
H = 128
G = 256
T = 10

NC = 2
NS = 16
EPT = E // (NC * NS)
CH = 125
NCH = EPT // CH
GRP = 40
NG = NCH // GRP
RPT = NP // NS


def _make_agg(nh: int):
    mesh = plsc.VectorSubcoreMesh(core_axis_name="c", subcore_axis_name="s")
    out_type = jax.ShapeDtypeStruct((NC, NP, H), jnp.float32)
    scratch = [
        pltpu.VMEM_SHARED((NP, H), jnp.float32),
        pltpu.VMEM((GRP, CH), jnp.int32),
        pltpu.VMEM((GRP, CH), jnp.int32),
        pltpu.VMEM((CH, H), jnp.float32),
        pltpu.VMEM((CH, H), jnp.float32),
        pltpu.SemaphoreType.DMA,
        pltpu.SemaphoreType.DMA,
        pltpu.SemaphoreType.DMA,
        pltpu.SemaphoreType.DMA,
    ]

    def body(h_hbm, src_hbm, dst_hbm, z2_hbm, agg_hbm,
             agg_sh, srcv, dstv, rows0, rows1, sem0, sem1, ses0, ses1):
        c = lax.axis_index("c")
        s = lax.axis_index("s")
        w = c * NS + s
        pltpu.sync_copy(z2_hbm, agg_sh.at[pl.ds(s * RPT, RPT)])
        plsc.subcore_barrier()

        def group(g, carry):
            r0 = w * NCH + g * GRP
            pltpu.sync_copy(src_hbm.at[pl.ds(r0, GRP)], srcv)
            pltpu.sync_copy(dst_hbm.at[pl.ds(r0, GRP)], dstv)
            pltpu.async_copy(h_hbm.at[srcv.at[0]], rows0, sem0)
            pltpu.async_copy(h_hbm.at[srcv.at[1]], rows1, sem1)

            def it(p, carry2):
                i = 2 * p
                pltpu.make_async_copy(h_hbm.at[srcv.at[i]], rows0, sem0).wait()
                pltpu.sync_copy(rows0, agg_sh.at[dstv.at[i]], add=True)
                pltpu.async_copy(h_hbm.at[srcv.at[i + 2]], rows0, sem0)
                pltpu.make_async_copy(h_hbm.at[srcv.at[i + 1]], rows1,
                                      sem1).wait()
                pltpu.sync_copy(rows1, agg_sh.at[dstv.at[i + 1]], add=True)
                pltpu.async_copy(h_hbm.at[srcv.at[i + 3]], rows1, sem1)
                return carry2

            lax.fori_loop(0, GRP // 2 - 1, it, 0)
            pltpu.make_async_copy(h_hbm.at[srcv.at[GRP - 2]], rows0,
                                  sem0).wait()
            pltpu.sync_copy(rows0, agg_sh.at[dstv.at[GRP - 2]], add=True)
            pltpu.make_async_copy(h_hbm.at[srcv.at[GRP - 1]], rows1,
                                  sem1).wait()
            pltpu.sync_copy(rows1, agg_sh.at[dstv.at[GRP - 1]], add=True)
            return carry

        lax.fori_loop(0, NG, group, 0)
        plsc.subcore_barrier()
        pltpu.sync_copy(agg_sh.at[pl.ds(s * RPT, RPT)],
                        agg_hbm.at[c, pl.ds(s * RPT, RPT)])

    return pl.kernel(body, mesh=mesh, out_type=out_type, scratch_types=scratch)


def _scatter_agg(h, src2d, dst2d):
    z2 = jnp.zeros((RPT, H), jnp.float32)
    return _make_agg(h.shape[0])(h, src2d, dst2d, z2)


def _make_count():
    mesh = plsc.VectorSubcoreMesh(core_axis_name="c", subcore_axis_name="s")
    out_type = jax.ShapeDtypeStruct((NC, NP, H), jnp.float32)
    KD = 8
    scratch = [
        pltpu.VMEM_SHARED((NP, H), jnp.float32),
        pltpu.VMEM((GRP, CH), jnp.int32),
        pltpu.VMEM((CH, H), jnp.float32),
        pltpu.SemaphoreType.DMA,
    ]

    def body(dst_hbm, z2_hbm, o2_hbm, cnt_hbm, cnt_sh, dstv, onesv, sem):
        c = lax.axis_index("c")
        s = lax.axis_index("s")
        w = c * NS + s
        pltpu.sync_copy(z2_hbm, cnt_sh.at[pl.ds(s * RPT, RPT)])
        pltpu.sync_copy(o2_hbm, onesv)
        plsc.subcore_barrier()

        def group(g, carry):
            pltpu.sync_copy(dst_hbm.at[pl.ds(w * NCH + g * GRP, GRP)], dstv)

            def it(q, carry2):
                for k in range(KD):
                    pltpu.async_copy(onesv, cnt_sh.at[dstv.at[q * KD + k]],
                                     sem, add=True)
                for k in range(KD):
                    pltpu.make_async_copy(onesv,
                                          cnt_sh.at[dstv.at[q * KD + k]],
                                          sem).wait()
                return carry2

            lax.fori_loop(0, GRP // KD, it, 0)
            return carry

        lax.fori_loop(0, NG, group, 0)
        plsc.subcore_barrier()
        pltpu.sync_copy(cnt_sh.at[pl.ds(s * RPT, RPT)],
                        cnt_hbm.at[c, pl.ds(s * RPT, RPT)])

    return pl.kernel(body, mesh=mesh, out_type=out_type, scratch_types=scratch)


def _count(dst2d):
    z2 = jnp.zeros((RPT, H), jnp.float32)
    o2 = jnp.ones((CH, H), jnp.float32)
    return _make_count()(dst2d, z2, o2)


def _mlp0_body(x_ref, wa_ref, wb_ref, o_ref):
    h = jnp.maximum(
        jnp.dot(x_ref[...], wa_ref[...], preferred_element_type=jnp.float32), 0.0)
    o_ref[...] = jnp.maximum(
        jnp.dot(h, wb_ref[...], preferred_element_type=jnp.float32), 0.0)


def _mlp0(x, wa, wb):
    n = x.shape[0]
    blk = 1000
    return pl.pallas_call(
        _mlp0_body,
        grid=(n // blk,),
        in_specs=[pl.BlockSpec((blk, H), lambda i: (i, 0)),
                  pl.BlockSpec((H, H), lambda i: (0, 0)),
                  pl.BlockSpec((H, H), lambda i: (0, 0))],
        out_specs=pl.BlockSpec((blk, H), lambda i: (i, 0)),
        out_shape=jax.ShapeDtypeStruct((n, H), jnp.float32),
    )(x, wa, wb)


def _mlp1_body(a_ref, c_ref, wa_ref, wb_ref, o_ref, inv_ref):
    cnt = c_ref[0, :, 0] + c_ref[1, :, 0]
    inv = 1.0 / jnp.maximum(cnt, 1.0)
    inv_ref[...] = inv[:, None]
    mean = (a_ref[0] + a_ref[1]) * inv[:, None]
    h = jnp.maximum(
        jnp.dot(mean, wa_ref[...], preferred_element_type=jnp.float32), 0.0)
    o_ref[...] = jnp.maximum(
        jnp.dot(h, wb_ref[...], preferred_element_type=jnp.float32), 0.0)


def _mlp1(a, c, wa, wb):
    blk = 1280
    return pl.pallas_call(
        _mlp1_body,
        grid=(NP // blk,),
        in_specs=[pl.BlockSpec((NC, blk, H), lambda i: (0, i, 0)),
                  pl.BlockSpec((NC, blk, H), lambda i: (0, i, 0)),
                  pl.BlockSpec((H, H), lambda i: (0, 0)),
                  pl.BlockSpec((H, H), lambda i: (0, 0))],
        out_specs=[pl.BlockSpec((blk, H), lambda i: (i, 0)),
                   pl.BlockSpec((blk, 1), lambda i: (i, 0))],
        out_shape=[jax.ShapeDtypeStruct((NP, H), jnp.float32),
                   jax.ShapeDtypeStruct((NP, 1), jnp.float32)],
    )(a, c, wa, wb)


def _mlp2_head_body(a_ref, inv_ref, wa_ref, wb_ref, b_ref,
                    wout_ref, bout_ref, o_ref, p_ref, psum, pcnt):
    i = pl.program_id(0)
    nb = pl.num_programs(0)

    @pl.when(i == 0)
    def _():
        psum[...] = jnp.zeros_like(psum)
        pcnt[...] = jnp.zeros_like(pcnt)

    mean = (a_ref[0] + a_ref[1]) * inv_ref[...]
    h = jnp.maximum(
        jnp.dot(mean, wa_ref[...], preferred_element_type=jnp.float32), 0.0)
    x3 = jnp.maximum(
        jnp.dot(h, wb_ref[...], preferred_element_type=jnp.float32), 0.0)
    o_ref[...] = x3

    b = b_ref[0, 0, :]
    blk = b.shape[0]
    gids = lax.broadcasted_iota(jnp.int32, (blk, G), 1)
    mask = (b[:, None] == gids).astype(jnp.float32)
    psum[...] += lax.dot_general(mask, x3,
                                 (((0,), (0,)), ((), ())),
                                 preferred_element_type=jnp.float32)
    pcnt[...] += lax.dot_general(mask, jnp.ones((blk, H), jnp.float32),
                                 (((0,), (0,)), ((), ())),
                                 preferred_element_type=jnp.float32)

    @pl.when(i == nb - 1)
    def _():
        pooled = psum[...] / jnp.maximum(pcnt[...], 1.0)
        logits = jnp.dot(pooled, wout_ref[...],
                         preferred_element_type=jnp.float32) + bout_ref[...]
        m = jnp.max(logits, axis=-1, keepdims=True)
        e = jnp.exp(logits - m)
        p = e / jnp.sum(e, axis=-1, keepdims=True)
        p_ref[...] = jnp.clip(p, 1e-8, 1.0)


def _mlp2_head(a, inv, wa, wb, batch3d, wout, bout2d):
    blk = 1280
    return pl.pallas_call(
        _mlp2_head_body,
        grid=(NP // blk,),
        in_specs=[pl.BlockSpec((NC, blk, H), lambda i: (0, i, 0)),
                  pl.BlockSpec((blk, 1), lambda i: (i, 0)),
                  pl.BlockSpec((H, H), lambda i: (0, 0)),
                  pl.BlockSpec((H, H), lambda i: (0, 0)),
                  pl.BlockSpec((1, 1, blk), lambda i: (i, 0, 0)),
                  pl.BlockSpec((H, T), lambda i: (0, 0)),
                  pl.BlockSpec((1, T), lambda i: (0, 0))],
        out_specs=[pl.BlockSpec((blk, H), lambda i: (i, 0)),
                   pl.BlockSpec((G, T), lambda i: (0, 0))],
        out_shape=[jax.ShapeDtypeStruct((NP, H), jnp.float32),
                   jax.ShapeDtypeStruct((G, T), jnp.float32)],
        scratch_shapes=[pltpu.VMEM((G, H), jnp.float32),
                        pltpu.VMEM((G, H), jnp.float32)],
    )(a, inv, wa, wb, batch3d, wout, bout2d)


def kernel(x, edge_index, edge_attr, batch, W0a, W0b, W1a, W1b, W2a, W2b,
           Wout, bout):
    src2d = edge_index[0].reshape(E // CH, CH)
    dst2d = edge_index[1].reshape(E // CH, CH)

    cnt = _count(dst2d)
    x1 = _mlp0(x, W0a, W0b)
    agg1 = _scatter_agg(x1, src2d, dst2d)
    x2p, inv = _mlp1(agg1, cnt, W1a, W1b)
    agg2 = _scatter_agg(x2p, src2d, dst2d)

    batch_pad = jnp.concatenate(
        [batch, jnp.full((NP - N,), G, jnp.int32)]).reshape(NP // 1280, 1, 1280)
    x3p, p = _mlp2_head(agg2, inv, W2a, W2b, batch_pad,
                        Wout, bout.reshape(1, T))

    node_embeddings = jnp.concatenate([x1, x2p[:N], x3p[:N]], axis=1)
    return (p, node_embeddings)


# speedup vs baseline: 1.0640x; 1.0100x over previous
import functools

import jax
import jax.numpy as jnp
from jax import lax
from jax.experimental import pallas as pl
from jax.experimental.pallas import tpu as pltpu
from jax.experimental.pallas import tpu_sc as plsc

N = 10000
NP = 10240
E = 320000
H = 128
G = 256
T = 10

NC = 2
NS = 16
EPT = E // (NC * NS)
CH = 125
NCH = EPT // CH
GRP = 40
NG = NCH // GRP
RPT = NP // NS


def _make_agg(nh: int):
    mesh = plsc.VectorSubcoreMesh(core_axis_name="c", subcore_axis_name="s")
    out_type = jax.ShapeDtypeStruct((NC, NP, H), jnp.float32)
    scratch = [
        pltpu.VMEM_SHARED((NP, H), jnp.float32),
        pltpu.VMEM((GRP, CH), jnp.int32),
        pltpu.VMEM((GRP, CH), jnp.int32),
        pltpu.VMEM((CH, H), jnp.float32),
        pltpu.VMEM((CH, H), jnp.float32),
        pltpu.SemaphoreType.DMA,
        pltpu.SemaphoreType.DMA,
        pltpu.SemaphoreType.DMA,
        pltpu.SemaphoreType.DMA,
    ]

    def body(h_hbm, src_hbm, dst_hbm, z2_hbm, agg_hbm,
             agg_sh, srcv, dstv, rows0, rows1, sem0, sem1, ses0, ses1):
        c = lax.axis_index("c")
        s = lax.axis_index("s")
        w = c * NS + s
        pltpu.sync_copy(z2_hbm, agg_sh.at[pl.ds(s * RPT, RPT)])
        plsc.subcore_barrier()

        def sl(i):
            return srcv.at[i]

        def group(g, carry):
            r0 = w * NCH + g * GRP
            pltpu.sync_copy(src_hbm.at[pl.ds(r0, GRP)], srcv)
            pltpu.sync_copy(dst_hbm.at[pl.ds(r0, GRP)], dstv)
            pltpu.async_copy(h_hbm.at[sl(0)], rows0, sem0)
            pltpu.async_copy(h_hbm.at[sl(1)], rows1, sem1)

            def it(p, carry2):
                i = 2 * p
                pltpu.make_async_copy(h_hbm.at[sl(i)], rows0, sem0).wait()
                pltpu.sync_copy(rows0, agg_sh.at[dstv.at[i]], add=True)
                pltpu.async_copy(h_hbm.at[sl(i + 2)], rows0, sem0)
                pltpu.make_async_copy(h_hbm.at[sl(i + 1)], rows1,
                                      sem1).wait()
                pltpu.sync_copy(rows1, agg_sh.at[dstv.at[i + 1]], add=True)
                pltpu.async_copy(h_hbm.at[sl(i + 3)], rows1, sem1)
                return carry2

            lax.fori_loop(0, GRP // 2 - 1, it, 0)
            pltpu.make_async_copy(h_hbm.at[sl(GRP - 2)], rows0,
                                  sem0).wait()
            pltpu.sync_copy(rows0, agg_sh.at[dstv.at[GRP - 2]], add=True)
            pltpu.make_async_copy(h_hbm.at[sl(GRP - 1)], rows1,
                                  sem1).wait()
            pltpu.sync_copy(rows1, agg_sh.at[dstv.at[GRP - 1]], add=True)
            return carry

        lax.fori_loop(0, NG, group, 0)
        plsc.subcore_barrier()
        pltpu.sync_copy(agg_sh.at[pl.ds(s * RPT, RPT)],
                        agg_hbm.at[c, pl.ds(s * RPT, RPT)])

    return pl.kernel(body, mesh=mesh, out_type=out_type, scratch_types=scratch)


def _scatter_agg(h, src2d, dst2d):
    z2 = jnp.zeros((RPT, H), jnp.float32)
    return _make_agg(h.shape[0])(h, src2d, dst2d, z2)


def _make_count():
    mesh = plsc.VectorSubcoreMesh(core_axis_name="c", subcore_axis_name="s")
    out_type = jax.ShapeDtypeStruct((NC, NP, H), jnp.float32)
    KD = 8
    scratch = [
        pltpu.VMEM_SHARED((NP, H), jnp.float32),
        pltpu.VMEM((GRP, CH), jnp.int32),
        pltpu.VMEM((CH, H), jnp.float32),
        pltpu.SemaphoreType.DMA,
    ]

    def body(dst_hbm, z2_hbm, o2_hbm, cnt_hbm, cnt_sh, dstv, onesv, sem):
        c = lax.axis_index("c")
        s = lax.axis_index("s")
        w = c * NS + s
        pltpu.sync_copy(z2_hbm, cnt_sh.at[pl.ds(s * RPT, RPT)])
        pltpu.sync_copy(o2_hbm, onesv)
        plsc.subcore_barrier()

        def group(g, carry):
            pltpu.sync_copy(dst_hbm.at[pl.ds(w * NCH + g * GRP, GRP)], dstv)

            def it(q, carry2):
                for k in range(KD):
                    pltpu.async_copy(onesv, cnt_sh.at[dstv.at[q * KD + k]],
                                     sem, add=True)
                for k in range(KD):
                    pltpu.make_async_copy(onesv,
                                          cnt_sh.at[dstv.at[q * KD + k]],
                                          sem).wait()
                return carry2

            lax.fori_loop(0, GRP // KD, it, 0)
            return carry

        lax.fori_loop(0, NG, group, 0)
        plsc.subcore_barrier()
        pltpu.sync_copy(cnt_sh.at[pl.ds(s * RPT, RPT)],
                        cnt_hbm.at[c, pl.ds(s * RPT, RPT)])

    return pl.kernel(body, mesh=mesh, out_type=out_type, scratch_types=scratch)


def _count(dst2d):
    z2 = jnp.zeros((RPT, H), jnp.float32)
    o2 = jnp.ones((CH, H), jnp.float32)
    return _make_count()(dst2d, z2, o2)


def _mlp0_body(x_ref, wa_ref, wb_ref, o_ref):
    h = jnp.maximum(
        jnp.dot(x_ref[...], wa_ref[...], preferred_element_type=jnp.float32), 0.0)
    o_ref[...] = jnp.maximum(
        jnp.dot(h, wb_ref[...], preferred_element_type=jnp.float32), 0.0)


def _mlp0(x, wa, wb):
    n = x.shape[0]
    blk = 2000
    return pl.pallas_call(
        _mlp0_body,
        grid=(n // blk,),
        in_specs=[pl.BlockSpec((blk, H), lambda i: (i, 0)),
                  pl.BlockSpec((H, H), lambda i: (0, 0)),
                  pl.BlockSpec((H, H), lambda i: (0, 0))],
        out_specs=pl.BlockSpec((blk, H), lambda i: (i, 0)),
        out_shape=jax.ShapeDtypeStruct((n, H), jnp.float32),
    )(x, wa, wb)


def _mlp1_body(a_ref, c_ref, wa_ref, wb_ref, o_ref, inv_ref):
    cnt = c_ref[0, :, 0] + c_ref[1, :, 0]
    inv = 1.0 / jnp.maximum(cnt, 1.0)
    inv_ref[...] = inv[:, None]
    mean = (a_ref[0] + a_ref[1]) * inv[:, None]
    h = jnp.maximum(
        jnp.dot(mean, wa_ref[...], preferred_element_type=jnp.float32), 0.0)
    o_ref[...] = jnp.maximum(
        jnp.dot(h, wb_ref[...], preferred_element_type=jnp.float32), 0.0)


def _mlp1(a, c, wa, wb):
    blk = 1280
    return pl.pallas_call(
        _mlp1_body,
        grid=(NP // blk,),
        in_specs=[pl.BlockSpec((NC, blk, H), lambda i: (0, i, 0)),
                  pl.BlockSpec((NC, blk, H), lambda i: (0, i, 0)),
                  pl.BlockSpec((H, H), lambda i: (0, 0)),
                  pl.BlockSpec((H, H), lambda i: (0, 0))],
        out_specs=[pl.BlockSpec((blk, H), lambda i: (i, 0)),
                   pl.BlockSpec((blk, 1), lambda i: (i, 0))],
        out_shape=[jax.ShapeDtypeStruct((NP, H), jnp.float32),
                   jax.ShapeDtypeStruct((NP, 1), jnp.float32)],
    )(a, c, wa, wb)


def _mlp2_head_body(a_ref, inv_ref, wa_ref, wb_ref, b_ref,
                    wout_ref, bout_ref, o_ref, p_ref, psum, pcnt):
    i = pl.program_id(0)
    nb = pl.num_programs(0)

    @pl.when(i == 0)
    def _():
        psum[...] = jnp.zeros_like(psum)
        pcnt[...] = jnp.zeros_like(pcnt)

    mean = (a_ref[0] + a_ref[1]) * inv_ref[...]
    h = jnp.maximum(
        jnp.dot(mean, wa_ref[...], preferred_element_type=jnp.float32), 0.0)
    x3 = jnp.maximum(
        jnp.dot(h, wb_ref[...], preferred_element_type=jnp.float32), 0.0)
    o_ref[...] = x3

    b = b_ref[0, 0, :]
    blk = b.shape[0]
    gids = lax.broadcasted_iota(jnp.int32, (blk, G), 1)
    mask = (b[:, None] == gids).astype(jnp.float32)
    psum[...] += lax.dot_general(mask, x3,
                                 (((0,), (0,)), ((), ())),
                                 preferred_element_type=jnp.float32)
    pcnt[...] += lax.dot_general(mask, jnp.ones((blk, H), jnp.float32),
                                 (((0,), (0,)), ((), ())),
                                 preferred_element_type=jnp.float32)

    @pl.when(i == nb - 1)
    def _():
        pooled = psum[...] / jnp.maximum(pcnt[...], 1.0)
        logits = jnp.dot(pooled, wout_ref[...],
                         preferred_element_type=jnp.float32) + bout_ref[...]
        m = jnp.max(logits, axis=-1, keepdims=True)
        e = jnp.exp(logits - m)
        p = e / jnp.sum(e, axis=-1, keepdims=True)
        p_ref[...] = jnp.clip(p, 1e-8, 1.0)


def _mlp2_head(a, inv, wa, wb, batch3d, wout, bout2d):
    blk = 1280
    return pl.pallas_call(
        _mlp2_head_body,
        grid=(NP // blk,),
        in_specs=[pl.BlockSpec((NC, blk, H), lambda i: (0, i, 0)),
                  pl.BlockSpec((blk, 1), lambda i: (i, 0)),
                  pl.BlockSpec((H, H), lambda i: (0, 0)),
                  pl.BlockSpec((H, H), lambda i: (0, 0)),
                  pl.BlockSpec((1, 1, blk), lambda i: (i, 0, 0)),
                  pl.BlockSpec((H, T), lambda i: (0, 0)),
                  pl.BlockSpec((1, T), lambda i: (0, 0))],
        out_specs=[pl.BlockSpec((blk, H), lambda i: (i, 0)),
                   pl.BlockSpec((G, T), lambda i: (0, 0))],
        out_shape=[jax.ShapeDtypeStruct((NP, H), jnp.float32),
                   jax.ShapeDtypeStruct((G, T), jnp.float32)],
        scratch_shapes=[pltpu.VMEM((G, H), jnp.float32),
                        pltpu.VMEM((G, H), jnp.float32)],
    )(a, inv, wa, wb, batch3d, wout, bout2d)


def kernel(x, edge_index, edge_attr, batch, W0a, W0b, W1a, W1b, W2a, W2b,
           Wout, bout):
    src1d = edge_index[0].reshape(E // CH, CH)
    dst2d = edge_index[1].reshape(E // CH, CH)

    cnt = _count(dst2d)
    x1 = _mlp0(x, W0a, W0b)
    dst2d_g, _ = lax.optimization_barrier((dst2d, cnt[0, 0, 0]))
    agg1 = _scatter_agg(x1, src1d, dst2d_g)
    x2p, inv = _mlp1(agg1, cnt, W1a, W1b)
    agg2 = _scatter_agg(x2p, src1d, dst2d)

    batch_pad = jnp.concatenate(
        [batch, jnp.full((NP - N,), G, jnp.int32)]).reshape(NP // 1280, 1, 1280)
    x3p, p = _mlp2_head(agg2, inv, W2a, W2b, batch_pad,
                        Wout, bout.reshape(1, T))

    node_embeddings = jnp.concatenate([x1, x2p[:N], x3p[:N]], axis=1)
    return (p, node_embeddings)


# speedup vs baseline: 1.0648x; 1.0008x over previous
import functools

import jax
import jax.numpy as jnp
from jax import lax
from jax.experimental import pallas as pl
from jax.experimental.pallas import tpu as pltpu
from jax.experimental.pallas import tpu_sc as plsc

N = 10000
NP = 10240
E = 320000
H = 128
G = 256
T = 10

NC = 2
NS = 16
EPT = E // (NC * NS)
CH = 125
NCH = EPT // CH
GRP = 40
NG = NCH // GRP
RPT = NP // NS


def _make_agg(nh: int):
    mesh = plsc.VectorSubcoreMesh(core_axis_name="c", subcore_axis_name="s")
    out_type = jax.ShapeDtypeStruct((NC, NP, H), jnp.float32)
    scratch = [
        pltpu.VMEM_SHARED((NP, H), jnp.float32),
        pltpu.VMEM((GRP, CH), jnp.int32),
        pltpu.VMEM((GRP, CH), jnp.int32),
        pltpu.VMEM((CH, H), jnp.float32),
        pltpu.VMEM((CH, H), jnp.float32),
        pltpu.SemaphoreType.DMA,
        pltpu.SemaphoreType.DMA,
        pltpu.SemaphoreType.DMA,
        pltpu.SemaphoreType.DMA,
    ]

    def body(h_hbm, src_hbm, dst_hbm, z2_hbm, agg_hbm,
             agg_sh, srcv, dstv, rows0, rows1, sem0, sem1, ses0, ses1):
        c = lax.axis_index("c")
        s = lax.axis_index("s")
        w = c * NS + s
        pltpu.sync_copy(z2_hbm, agg_sh.at[pl.ds(s * RPT, RPT)])
        plsc.subcore_barrier()

        def sl(i):
            return srcv.at[i]

        def group(g, carry):
            r0 = w * NCH + g * GRP
            pltpu.sync_copy(src_hbm.at[pl.ds(r0, GRP)], srcv)
            pltpu.sync_copy(dst_hbm.at[pl.ds(r0, GRP)], dstv)
            pltpu.async_copy(h_hbm.at[sl(0)], rows0, sem0)
            pltpu.async_copy(h_hbm.at[sl(1)], rows1, sem1)

            def it(p, carry2):
                i = 2 * p
                pltpu.make_async_copy(h_hbm.at[sl(i)], rows0, sem0).wait()
                pltpu.sync_copy(rows0, agg_sh.at[dstv.at[i]], add=True)
                pltpu.async_copy(h_hbm.at[sl(i + 2)], rows0, sem0)
                pltpu.make_async_copy(h_hbm.at[sl(i + 1)], rows1,
                                      sem1).wait()
                pltpu.sync_copy(rows1, agg_sh.at[dstv.at[i + 1]], add=True)
                pltpu.async_copy(h_hbm.at[sl(i + 3)], rows1, sem1)
                return carry2

            lax.fori_loop(0, GRP // 2 - 1, it, 0)
            pltpu.make_async_copy(h_hbm.at[sl(GRP - 2)], rows0,
                                  sem0).wait()
            pltpu.sync_copy(rows0, agg_sh.at[dstv.at[GRP - 2]], add=True)
            pltpu.make_async_copy(h_hbm.at[sl(GRP - 1)], rows1,
                                  sem1).wait()
            pltpu.sync_copy(rows1, agg_sh.at[dstv.at[GRP - 1]], add=True)
            return carry

        lax.fori_loop(0, NG, group, 0)
        plsc.subcore_barrier()
        pltpu.sync_copy(agg_sh.at[pl.ds(s * RPT, RPT)],
                        agg_hbm.at[c, pl.ds(s * RPT, RPT)])

    return pl.kernel(body, mesh=mesh, out_type=out_type, scratch_types=scratch)


def _scatter_agg(h, src2d, dst2d):
    z2 = jnp.zeros((RPT, H), jnp.float32)
    return _make_agg(h.shape[0])(h, src2d, dst2d, z2)


def _make_count():
    mesh = plsc.VectorSubcoreMesh(core_axis_name="c", subcore_axis_name="s")
    out_type = jax.ShapeDtypeStruct((NC, NP, H), jnp.float32)
    KD = 16
    scratch = [
        pltpu.VMEM_SHARED((NP, H), jnp.float32),
        pltpu.VMEM((NCH, CH), jnp.int32),
        pltpu.VMEM((CH, H), jnp.float32),
        pltpu.SemaphoreType.DMA,
    ]

    def body(dst_hbm, z2_hbm, o2_hbm, cnt_hbm, cnt_sh, dstv, onesv, sem):
        c = lax.axis_index("c")
        s = lax.axis_index("s")
        w = c * NS + s
        pltpu.sync_copy(dst_hbm.at[pl.ds(w * NCH, NCH)], dstv)
        pltpu.sync_copy(z2_hbm, cnt_sh.at[pl.ds(s * RPT, RPT)])
        pltpu.sync_copy(o2_hbm, onesv)
        plsc.subcore_barrier()

        def it(q, carry2):
            for k in range(KD):
                pltpu.async_copy(onesv, cnt_sh.at[dstv.at[q * KD + k]],
                                 sem, add=True)
            for k in range(KD):
                pltpu.make_async_copy(onesv,
                                      cnt_sh.at[dstv.at[q * KD + k]],
                                      sem).wait()
            return carry2

        lax.fori_loop(0, NCH // KD, it, 0)
        plsc.subcore_barrier()
        pltpu.sync_copy(cnt_sh.at[pl.ds(s * RPT, RPT)],
                        cnt_hbm.at[c, pl.ds(s * RPT, RPT)])

    return pl.kernel(body, mesh=mesh, out_type=out_type, scratch_types=scratch)


def _count(dst2d):
    z2 = jnp.zeros((RPT, H), jnp.float32)
    o2 = jnp.ones((CH, H), jnp.float32)
    return _make_count()(dst2d, z2, o2)


def _mlp0_body(x_ref, wa_ref, wb_ref, o_ref):
    h = jnp.maximum(
        jnp.dot(x_ref[...], wa_ref[...], preferred_element_type=jnp.float32), 0.0)
    o_ref[...] = jnp.maximum(
        jnp.dot(h, wb_ref[...], preferred_element_type=jnp.float32), 0.0)


def _mlp0(x, wa, wb):
    n = x.shape[0]
    blk = 2000
    return pl.pallas_call(
        _mlp0_body,
        grid=(n // blk,),
        in_specs=[pl.BlockSpec((blk, H), lambda i: (i, 0)),
                  pl.BlockSpec((H, H), lambda i: (0, 0)),
                  pl.BlockSpec((H, H), lambda i: (0, 0))],
        out_specs=pl.BlockSpec((blk, H), lambda i: (i, 0)),
        out_shape=jax.ShapeDtypeStruct((n, H), jnp.float32),
    )(x, wa, wb)


def _mlp1_body(a_ref, c_ref, wa_ref, wb_ref, o_ref, inv_ref):
    cnt = c_ref[0, :, 0] + c_ref[1, :, 0]
    inv = 1.0 / jnp.maximum(cnt, 1.0)
    inv_ref[...] = inv[:, None]
    mean = (a_ref[0] + a_ref[1]) * inv[:, None]
    h = jnp.maximum(
        jnp.dot(mean, wa_ref[...], preferred_element_type=jnp.float32), 0.0)
    o_ref[...] = jnp.maximum(
        jnp.dot(h, wb_ref[...], preferred_element_type=jnp.float32), 0.0)


def _mlp1(a, c, wa, wb):
    blk = 1280
    return pl.pallas_call(
        _mlp1_body,
        grid=(NP // blk,),
        in_specs=[pl.BlockSpec((NC, blk, H), lambda i: (0, i, 0)),
                  pl.BlockSpec((NC, blk, H), lambda i: (0, i, 0)),
                  pl.BlockSpec((H, H), lambda i: (0, 0)),
                  pl.BlockSpec((H, H), lambda i: (0, 0))],
        out_specs=[pl.BlockSpec((blk, H), lambda i: (i, 0)),
                   pl.BlockSpec((blk, 1), lambda i: (i, 0))],
        out_shape=[jax.ShapeDtypeStruct((NP, H), jnp.float32),
                   jax.ShapeDtypeStruct((NP, 1), jnp.float32)],
    )(a, c, wa, wb)


def _mlp2_head_body(a_ref, inv_ref, wa_ref, wb_ref, b_ref,
                    wout_ref, bout_ref, o_ref, p_ref, psum, pcnt):
    i = pl.program_id(0)
    nb = pl.num_programs(0)

    @pl.when(i == 0)
    def _():
        psum[...] = jnp.zeros_like(psum)
        pcnt[...] = jnp.zeros_like(pcnt)

    mean = (a_ref[0] + a_ref[1]) * inv_ref[...]
    h = jnp.maximum(
        jnp.dot(mean, wa_ref[...], preferred_element_type=jnp.float32), 0.0)
    x3 = jnp.maximum(
        jnp.dot(h, wb_ref[...], preferred_element_type=jnp.float32), 0.0)
    o_ref[...] = x3

    b = b_ref[0, 0, :]
    blk = b.shape[0]
    gids = lax.broadcasted_iota(jnp.int32, (blk, G), 1)
    mask = (b[:, None] == gids).astype(jnp.float32)
    psum[...] += lax.dot_general(mask, x3,
                                 (((0,), (0,)), ((), ())),
                                 preferred_element_type=jnp.float32)
    pcnt[...] += lax.dot_general(mask, jnp.ones((blk, H), jnp.float32),
                                 (((0,), (0,)), ((), ())),
                                 preferred_element_type=jnp.float32)

    @pl.when(i == nb - 1)
    def _():
        pooled = psum[...] / jnp.maximum(pcnt[...], 1.0)
        logits = jnp.dot(pooled, wout_ref[...],
                         preferred_element_type=jnp.float32) + bout_ref[...]
        m = jnp.max(logits, axis=-1, keepdims=True)
        e = jnp.exp(logits - m)
        p = e / jnp.sum(e, axis=-1, keepdims=True)
        p_ref[...] = jnp.clip(p, 1e-8, 1.0)


def _mlp2_head(a, inv, wa, wb, batch3d, wout, bout2d):
    blk = 1280
    return pl.pallas_call(
        _mlp2_head_body,
        grid=(NP // blk,),
        in_specs=[pl.BlockSpec((NC, blk, H), lambda i: (0, i, 0)),
                  pl.BlockSpec((blk, 1), lambda i: (i, 0)),
                  pl.BlockSpec((H, H), lambda i: (0, 0)),
                  pl.BlockSpec((H, H), lambda i: (0, 0)),
                  pl.BlockSpec((1, 1, blk), lambda i: (i, 0, 0)),
                  pl.BlockSpec((H, T), lambda i: (0, 0)),
                  pl.BlockSpec((1, T), lambda i: (0, 0))],
        out_specs=[pl.BlockSpec((blk, H), lambda i: (i, 0)),
                   pl.BlockSpec((G, T), lambda i: (0, 0))],
        out_shape=[jax.ShapeDtypeStruct((NP, H), jnp.float32),
                   jax.ShapeDtypeStruct((G, T), jnp.float32)],
        scratch_shapes=[pltpu.VMEM((G, H), jnp.float32),
                        pltpu.VMEM((G, H), jnp.float32)],
    )(a, inv, wa, wb, batch3d, wout, bout2d)


def kernel(x, edge_index, edge_attr, batch, W0a, W0b, W1a, W1b, W2a, W2b,
           Wout, bout):
    src1d = edge_index[0].reshape(E // CH, CH)
    dst2d = edge_index[1].reshape(E // CH, CH)

    cnt = _count(dst2d)
    x1 = _mlp0(x, W0a, W0b)
    dst2d_g, _ = lax.optimization_barrier((dst2d, cnt[0, 0, 0]))
    agg1 = _scatter_agg(x1, src1d, dst2d_g)
    x2p, inv = _mlp1(agg1, cnt, W1a, W1b)
    agg2 = _scatter_agg(x2p, src1d, dst2d)

    batch_pad = jnp.concatenate(
        [batch, jnp.full((NP - N,), G, jnp.int32)]).reshape(NP // 1280, 1, 1280)
    x3p, p = _mlp2_head(agg2, inv, W2a, W2b, batch_pad,
                        Wout, bout.reshape(1, T))

    node_embeddings = jnp.concatenate([x1, x2p[:N], x3p[:N]], axis=1)
    return (p, node_embeddings)


# speedup vs baseline: 1.0656x; 1.0007x over previous
import jax
import jax.numpy as jnp
from jax import lax
from jax.experimental import pallas as pl
from jax.experimental.pallas import tpu as pltpu
from jax.experimental.pallas import tpu_sc as plsc

N = 10000
NP = 10240
E = 320000
H = 128
G = 256
T = 10

NC = 2
NS = 16
EPT = E // (NC * NS)
CH = 125
NCH = EPT // CH
GRP = 40
NG = NCH // GRP
RPT = NP // NS


def _make_agg(nh: int):
    mesh = plsc.VectorSubcoreMesh(core_axis_name="c", subcore_axis_name="s")
    out_type = jax.ShapeDtypeStruct((NC, NP, H), jnp.float32)
    scratch = [
        pltpu.VMEM_SHARED((NP, H), jnp.float32),
        pltpu.VMEM((GRP, CH), jnp.int32),
        pltpu.VMEM((GRP, CH), jnp.int32),
        pltpu.VMEM((CH, H), jnp.float32),
        pltpu.VMEM((CH, H), jnp.float32),
        pltpu.SemaphoreType.DMA,
        pltpu.SemaphoreType.DMA,
        pltpu.SemaphoreType.DMA,
        pltpu.SemaphoreType.DMA,
    ]

    def body(h_hbm, src_hbm, dst_hbm, z2_hbm, agg_hbm,
             agg_sh, srcv, dstv, rows0, rows1, sem0, sem1, ses0, ses1):
        c = lax.axis_index("c")
        s = lax.axis_index("s")
        w = c * NS + s
        pltpu.sync_copy(z2_hbm, agg_sh.at[pl.ds(s * RPT, RPT)])
        plsc.subcore_barrier()

        def sl(i):
            return srcv.at[i]

        def group(g, carry):
            r0 = w * NCH + g * GRP
            pltpu.sync_copy(src_hbm.at[pl.ds(r0, GRP)], srcv)
            pltpu.sync_copy(dst_hbm.at[pl.ds(r0, GRP)], dstv)
            pltpu.async_copy(h_hbm.at[sl(0)], rows0, sem0)
            pltpu.async_copy(h_hbm.at[sl(1)], rows1, sem1)

            def it(p, carry2):
                i = 2 * p
                pltpu.make_async_copy(h_hbm.at[sl(i)], rows0, sem0).wait()
                pltpu.sync_copy(rows0, agg_sh.at[dstv.at[i]], add=True)
                pltpu.async_copy(h_hbm.at[sl(i + 2)], rows0, sem0)
                pltpu.make_async_copy(h_hbm.at[sl(i + 1)], rows1,
                                      sem1).wait()
                pltpu.sync_copy(rows1, agg_sh.at[dstv.at[i + 1]], add=True)
                pltpu.async_copy(h_hbm.at[sl(i + 3)], rows1, sem1)
                return carry2

            lax.fori_loop(0, GRP // 2 - 1, it, 0)
            pltpu.make_async_copy(h_hbm.at[sl(GRP - 2)], rows0,
                                  sem0).wait()
            pltpu.sync_copy(rows0, agg_sh.at[dstv.at[GRP - 2]], add=True)
            pltpu.make_async_copy(h_hbm.at[sl(GRP - 1)], rows1,
                                  sem1).wait()
            pltpu.sync_copy(rows1, agg_sh.at[dstv.at[GRP - 1]], add=True)
            return carry

        lax.fori_loop(0, NG, group, 0)
        plsc.subcore_barrier()
        pltpu.sync_copy(agg_sh.at[pl.ds(s * RPT, RPT)],
                        agg_hbm.at[c, pl.ds(s * RPT, RPT)])

    return pl.kernel(body, mesh=mesh, out_type=out_type, scratch_types=scratch)


def _scatter_agg(h, src2d, dst2d):
    z2 = jnp.zeros((RPT, H), jnp.float32)
    return _make_agg(h.shape[0])(h, src2d, dst2d, z2)


def _make_count():
    mesh = plsc.VectorSubcoreMesh(core_axis_name="c", subcore_axis_name="s")
    out_type = jax.ShapeDtypeStruct((NC, NP, H), jnp.float32)
    KD = 16
    scratch = [
        pltpu.VMEM_SHARED((NP, H), jnp.float32),
        pltpu.VMEM((NCH, CH), jnp.int32),
        pltpu.VMEM((CH, H), jnp.float32),
        pltpu.SemaphoreType.DMA,
    ]

    def body(dst_hbm, z2_hbm, o2_hbm, cnt_hbm, cnt_sh, dstv, onesv, sem):
        c = lax.axis_index("c")
        s = lax.axis_index("s")
        w = c * NS + s
        pltpu.sync_copy(dst_hbm.at[pl.ds(w * NCH, NCH)], dstv)
        pltpu.sync_copy(z2_hbm, cnt_sh.at[pl.ds(s * RPT, RPT)])
        pltpu.sync_copy(o2_hbm, onesv)
        plsc.subcore_barrier()

        def it(q, carry2):
            for k in range(KD):
                pltpu.async_copy(onesv, cnt_sh.at[dstv.at[q * KD + k]],
                                 sem, add=True)
            for k in range(KD):
                pltpu.make_async_copy(onesv,
                                      cnt_sh.at[dstv.at[q * KD + k]],
                                      sem).wait()
            return carry2

        lax.fori_loop(0, NCH // KD, it, 0)
        plsc.subcore_barrier()
        pltpu.sync_copy(cnt_sh.at[pl.ds(s * RPT, RPT)],
                        cnt_hbm.at[c, pl.ds(s * RPT, RPT)])

    return pl.kernel(body, mesh=mesh, out_type=out_type, scratch_types=scratch)


def _count(dst2d):
    z2 = jnp.zeros((RPT, H), jnp.float32)
    o2 = jnp.ones((CH, H), jnp.float32)
    return _make_count()(dst2d, z2, o2)


def _mlp0_body(x_ref, wa_ref, wb_ref, o_ref):
    h = jnp.maximum(
        jnp.dot(x_ref[...], wa_ref[...], preferred_element_type=jnp.float32), 0.0)
    o_ref[...] = jnp.maximum(
        jnp.dot(h, wb_ref[...], preferred_element_type=jnp.float32), 0.0)


def _mlp0(x, wa, wb):
    n = x.shape[0]
    blk = 2000
    return pl.pallas_call(
        _mlp0_body,
        grid=(n // blk,),
        in_specs=[pl.BlockSpec((blk, H), lambda i: (i, 0)),
                  pl.BlockSpec((H, H), lambda i: (0, 0)),
                  pl.BlockSpec((H, H), lambda i: (0, 0))],
        out_specs=pl.BlockSpec((blk, H), lambda i: (i, 0)),
        out_shape=jax.ShapeDtypeStruct((n, H), jnp.float32),
    )(x, wa, wb)


def _mlp1_body(a_ref, c_ref, wa_ref, wb_ref, o_ref, inv_ref):
    cnt = c_ref[0, :, 0] + c_ref[1, :, 0]
    inv = 1.0 / jnp.maximum(cnt, 1.0)
    inv_ref[...] = inv[:, None]
    mean = (a_ref[0] + a_ref[1]) * inv[:, None]
    h = jnp.maximum(
        jnp.dot(mean, wa_ref[...], preferred_element_type=jnp.float32), 0.0)
    o_ref[...] = jnp.maximum(
        jnp.dot(h, wb_ref[...], preferred_element_type=jnp.float32), 0.0)


def _mlp1(a, c, wa, wb):
    blk = 1280
    return pl.pallas_call(
        _mlp1_body,
        grid=(NP // blk,),
        in_specs=[pl.BlockSpec((NC, blk, H), lambda i: (0, i, 0)),
                  pl.BlockSpec((NC, blk, H), lambda i: (0, i, 0)),
                  pl.BlockSpec((H, H), lambda i: (0, 0)),
                  pl.BlockSpec((H, H), lambda i: (0, 0))],
        out_specs=[pl.BlockSpec((blk, H), lambda i: (i, 0)),
                   pl.BlockSpec((blk, 1), lambda i: (i, 0))],
        out_shape=[jax.ShapeDtypeStruct((NP, H), jnp.float32),
                   jax.ShapeDtypeStruct((NP, 1), jnp.float32)],
    )(a, c, wa, wb)


def _mlp2_head_body(a_ref, inv_ref, wa_ref, wb_ref, b_ref,
                    wout_ref, bout_ref, o_ref, p_ref, psum, pcnt):
    i = pl.program_id(0)
    nb = pl.num_programs(0)

    @pl.when(i == 0)
    def _():
        psum[...] = jnp.zeros_like(psum)
        pcnt[...] = jnp.zeros_like(pcnt)

    mean = (a_ref[0] + a_ref[1]) * inv_ref[...]
    h = jnp.maximum(
        jnp.dot(mean, wa_ref[...], preferred_element_type=jnp.float32), 0.0)
    x3 = jnp.maximum(
        jnp.dot(h, wb_ref[...], preferred_element_type=jnp.float32), 0.0)
    o_ref[...] = x3

    b = b_ref[0, 0, :]
    blk = b.shape[0]
    gids = lax.broadcasted_iota(jnp.int32, (blk, G), 1)
    mask = (b[:, None] == gids).astype(jnp.float32)
    psum[...] += lax.dot_general(mask, x3,
                                 (((0,), (0,)), ((), ())),
                                 preferred_element_type=jnp.float32)
    pcnt[...] += lax.dot_general(mask, jnp.ones((blk, H), jnp.float32),
                                 (((0,), (0,)), ((), ())),
                                 preferred_element_type=jnp.float32)

    @pl.when(i == nb - 1)
    def _():
        pooled = psum[...] / jnp.maximum(pcnt[...], 1.0)
        logits = jnp.dot(pooled, wout_ref[...],
                         preferred_element_type=jnp.float32) + bout_ref[...]
        m = jnp.max(logits, axis=-1, keepdims=True)
        e = jnp.exp(logits - m)
        p = e / jnp.sum(e, axis=-1, keepdims=True)
        p_ref[...] = jnp.clip(p, 1e-8, 1.0)


def _mlp2_head(a, inv, wa, wb, batch3d, wout, bout2d):
    blk = 1280
    return pl.pallas_call(
        _mlp2_head_body,
        grid=(NP // blk,),
        in_specs=[pl.BlockSpec((NC, blk, H), lambda i: (0, i, 0)),
                  pl.BlockSpec((blk, 1), lambda i: (i, 0)),
                  pl.BlockSpec((H, H), lambda i: (0, 0)),
                  pl.BlockSpec((H, H), lambda i: (0, 0)),
                  pl.BlockSpec((1, 1, blk), lambda i: (i, 0, 0)),
                  pl.BlockSpec((H, T), lambda i: (0, 0)),
                  pl.BlockSpec((1, T), lambda i: (0, 0))],
        out_specs=[pl.BlockSpec((blk, H), lambda i: (i, 0)),
                   pl.BlockSpec((G, T), lambda i: (0, 0))],
        out_shape=[jax.ShapeDtypeStruct((NP, H), jnp.float32),
                   jax.ShapeDtypeStruct((G, T), jnp.float32)],
        scratch_shapes=[pltpu.VMEM((G, H), jnp.float32),
                        pltpu.VMEM((G, H), jnp.float32)],
    )(a, inv, wa, wb, batch3d, wout, bout2d)


def kernel(x, edge_index, edge_attr, batch, W0a, W0b, W1a, W1b, W2a, W2b,
           Wout, bout):
    src1d = edge_index[0].reshape(E // CH, CH)
    dst2d = edge_index[1].reshape(E // CH, CH)

    cnt = _count(dst2d)
    x1 = _mlp0(x, W0a, W0b)
    dst2d_g, _ = lax.optimization_barrier((dst2d, cnt[0, 0, 0]))
    agg1 = _scatter_agg(x1, src1d, dst2d_g)
    x2p, inv = _mlp1(agg1, cnt, W1a, W1b)
    agg2 = _scatter_agg(x2p, src1d, dst2d)

    batch_pad = jnp.concatenate(
        [batch, jnp.full((NP - N,), G, jnp.int32)]).reshape(NP // 1280, 1, 1280)
    x3p, p = _mlp2_head(agg2, inv, W2a, W2b, batch_pad,
                        Wout, bout.reshape(1, T))

    node_embeddings = jnp.concatenate([x1, x2p[:N], x3p[:N]], axis=1)
    return (p, node_embeddings)


# speedup vs baseline: 1.0672x; 1.0015x over previous
import jax
import jax.numpy as jnp
from jax import lax
from jax.experimental import pallas as pl
from jax.experimental.pallas import tpu as pltpu
from jax.experimental.pallas import tpu_sc as plsc

N = 10000
NP = 10240
E = 320000
H = 128
G = 256
T = 10

NC = 2
NS = 16
EPT = E // (NC * NS)
CH = 125
NCH = EPT // CH
GRP = 40
NG = NCH // GRP
RPT = NP // NS


def _make_agg(nh: int):
    mesh = plsc.VectorSubcoreMesh(core_axis_name="c", subcore_axis_name="s")
    out_type = jax.ShapeDtypeStruct((NC, NP, H), jnp.float32)
    scratch = [
        pltpu.VMEM_SHARED((NP, H), jnp.float32),
        pltpu.VMEM((GRP, CH), jnp.int32),
        pltpu.VMEM((GRP, CH), jnp.int32),
        pltpu.VMEM((CH, H), jnp.float32),
        pltpu.VMEM((CH, H), jnp.float32),
        pltpu.SemaphoreType.DMA,
        pltpu.SemaphoreType.DMA,
        pltpu.SemaphoreType.DMA,
        pltpu.SemaphoreType.DMA,
    ]

    def body(h_hbm, src_hbm, dst_hbm, z2_hbm, agg_hbm,
             agg_sh, srcv, dstv, rows0, rows1, sem0, sem1, ses0, ses1):
        c = lax.axis_index("c")
        s = lax.axis_index("s")
        w = c * NS + s
        pltpu.sync_copy(z2_hbm, agg_sh.at[pl.ds(s * RPT, RPT)])
        plsc.subcore_barrier()

        def sl(i):
            return srcv.at[i]

        def group(g, carry):
            r0 = w * NCH + g * GRP
            pltpu.sync_copy(src_hbm.at[pl.ds(r0, GRP)], srcv)
            pltpu.sync_copy(dst_hbm.at[pl.ds(r0, GRP)], dstv)
            pltpu.async_copy(h_hbm.at[sl(0)], rows0, sem0)
            pltpu.async_copy(h_hbm.at[sl(1)], rows1, sem1)

            def it(p, carry2):
                i = 2 * p
                pltpu.make_async_copy(h_hbm.at[sl(i)], rows0, sem0).wait()
                pltpu.sync_copy(rows0, agg_sh.at[dstv.at[i]], add=True)
                pltpu.async_copy(h_hbm.at[sl(i + 2)], rows0, sem0)
                pltpu.make_async_copy(h_hbm.at[sl(i + 1)], rows1,
                                      sem1).wait()
                pltpu.sync_copy(rows1, agg_sh.at[dstv.at[i + 1]], add=True)
                pltpu.async_copy(h_hbm.at[sl(i + 3)], rows1, sem1)
                return carry2

            lax.fori_loop(0, GRP // 2 - 1, it, 0)
            pltpu.make_async_copy(h_hbm.at[sl(GRP - 2)], rows0,
                                  sem0).wait()
            pltpu.sync_copy(rows0, agg_sh.at[dstv.at[GRP - 2]], add=True)
            pltpu.make_async_copy(h_hbm.at[sl(GRP - 1)], rows1,
                                  sem1).wait()
            pltpu.sync_copy(rows1, agg_sh.at[dstv.at[GRP - 1]], add=True)
            return carry

        lax.fori_loop(0, NG, group, 0)
        plsc.subcore_barrier()
        pltpu.sync_copy(agg_sh.at[pl.ds(s * RPT, RPT)],
                        agg_hbm.at[c, pl.ds(s * RPT, RPT)])

    return pl.kernel(body, mesh=mesh, out_type=out_type, scratch_types=scratch)


def _scatter_agg(h, src2d, dst2d):
    z2 = jnp.zeros((RPT, H), jnp.float32)
    return _make_agg(h.shape[0])(h, src2d, dst2d, z2)


def _make_count():
    mesh = plsc.VectorSubcoreMesh(core_axis_name="c", subcore_axis_name="s")
    out_type = jax.ShapeDtypeStruct((NC, NP, H), jnp.float32)
    KD = 16
    scratch = [
        pltpu.VMEM_SHARED((NP, H), jnp.float32),
        pltpu.VMEM((NCH, CH), jnp.int32),
        pltpu.VMEM((CH, H), jnp.float32),
        pltpu.SemaphoreType.DMA,
    ]

    def body(dst_hbm, z2_hbm, o2_hbm, cnt_hbm, cnt_sh, dstv, onesv, sem):
        c = lax.axis_index("c")
        s = lax.axis_index("s")
        w = c * NS + s
        pltpu.sync_copy(dst_hbm.at[pl.ds(w * NCH, NCH)], dstv)
        pltpu.sync_copy(z2_hbm, cnt_sh.at[pl.ds(s * RPT, RPT)])
        pltpu.sync_copy(o2_hbm, onesv)
        plsc.subcore_barrier()

        def it(q, carry2):
            for k in range(KD):
                pltpu.async_copy(onesv, cnt_sh.at[dstv.at[q * KD + k]],
                                 sem, add=True)
            for k in range(KD):
                pltpu.make_async_copy(onesv,
                                      cnt_sh.at[dstv.at[q * KD + k]],
                                      sem).wait()
            return carry2

        lax.fori_loop(0, NCH // KD, it, 0)
        plsc.subcore_barrier()
        pltpu.sync_copy(cnt_sh.at[pl.ds(s * RPT, RPT)],
                        cnt_hbm.at[c, pl.ds(s * RPT, RPT)])

    return pl.kernel(body, mesh=mesh, out_type=out_type, scratch_types=scratch)


def _count(dst2d):
    z2 = jnp.zeros((RPT, H), jnp.float32)
    o2 = jnp.ones((CH, H), jnp.float32)
    return _make_count()(dst2d, z2, o2)


def _mlp0_body(x_ref, wa_ref, wb_ref, o_ref):
    h = jnp.maximum(
        jnp.dot(x_ref[...], wa_ref[...], preferred_element_type=jnp.float32), 0.0)
    o_ref[...] = jnp.maximum(
        jnp.dot(h, wb_ref[...], preferred_element_type=jnp.float32), 0.0)


def _mlp0(x, wa, wb):
    n = x.shape[0]
    blk = 2000
    return pl.pallas_call(
        _mlp0_body,
        grid=(n // blk,),
        in_specs=[pl.BlockSpec((blk, H), lambda i: (i, 0)),
                  pl.BlockSpec((H, H), lambda i: (0, 0)),
                  pl.BlockSpec((H, H), lambda i: (0, 0))],
        out_specs=pl.BlockSpec((blk, H), lambda i: (i, 0)),
        out_shape=jax.ShapeDtypeStruct((n, H), jnp.float32),
    )(x, wa, wb)


def _mlp1_body(a_ref, c_ref, wa_ref, wb_ref, o_ref, inv_ref):
    cnt = c_ref[0, :, 0] + c_ref[1, :, 0]
    inv = 1.0 / jnp.maximum(cnt, 1.0)
    inv_ref[...] = inv[:, None]
    mean = (a_ref[0] + a_ref[1]) * inv[:, None]
    h = jnp.maximum(
        jnp.dot(mean, wa_ref[...], preferred_element_type=jnp.float32), 0.0)
    o_ref[...] = jnp.maximum(
        jnp.dot(h, wb_ref[...], preferred_element_type=jnp.float32), 0.0)


def _mlp1(a, c, wa, wb):
    blk = 1280
    return pl.pallas_call(
        _mlp1_body,
        grid=(NP // blk,),
        in_specs=[pl.BlockSpec((NC, blk, H), lambda i: (0, i, 0)),
                  pl.BlockSpec((NC, blk, H), lambda i: (0, i, 0)),
                  pl.BlockSpec((H, H), lambda i: (0, 0)),
                  pl.BlockSpec((H, H), lambda i: (0, 0))],
        out_specs=[pl.BlockSpec((blk, H), lambda i: (i, 0)),
                   pl.BlockSpec((blk, 1), lambda i: (i, 0))],
        out_shape=[jax.ShapeDtypeStruct((NP, H), jnp.float32),
                   jax.ShapeDtypeStruct((NP, 1), jnp.float32)],
    )(a, c, wa, wb)


def _mlp2_head_body(a_ref, inv_ref, wa_ref, wb_ref, b_ref,
                    wout_ref, bout_ref, o_ref, p_ref, psum, pcnt):
    i = pl.program_id(0)
    nb = pl.num_programs(0)

    @pl.when(i == 0)
    def _():
        psum[...] = jnp.zeros_like(psum)
        pcnt[...] = jnp.zeros_like(pcnt)

    mean = (a_ref[0] + a_ref[1]) * inv_ref[...]
    h = jnp.maximum(
        jnp.dot(mean, wa_ref[...], preferred_element_type=jnp.float32), 0.0)
    x3 = jnp.maximum(
        jnp.dot(h, wb_ref[...], preferred_element_type=jnp.float32), 0.0)
    o_ref[...] = x3

    b = b_ref[0, 0, :]
    blk = b.shape[0]
    gids = lax.broadcasted_iota(jnp.int32, (blk, G), 1)
    mask = (b[:, None] == gids).astype(jnp.float32)
    psum[...] += lax.dot_general(mask, x3,
                                 (((0,), (0,)), ((), ())),
                                 preferred_element_type=jnp.float32)
    pcnt[...] += lax.dot_general(mask, jnp.ones((blk, H), jnp.float32),
                                 (((0,), (0,)), ((), ())),
                                 preferred_element_type=jnp.float32)

    @pl.when(i == nb - 1)
    def _():
        pooled = psum[...] / jnp.maximum(pcnt[...], 1.0)
        logits = jnp.dot(pooled, wout_ref[...],
                         preferred_element_type=jnp.float32) + bout_ref[...]
        m = jnp.max(logits, axis=-1, keepdims=True)
        e = jnp.exp(logits - m)
        p = e / jnp.sum(e, axis=-1, keepdims=True)
        p_ref[...] = jnp.clip(p, 1e-8, 1.0)


def _mlp2_head(a, inv, wa, wb, batch3d, wout, bout2d):
    blk = 1280
    return pl.pallas_call(
        _mlp2_head_body,
        grid=(NP // blk,),
        in_specs=[pl.BlockSpec((NC, blk, H), lambda i: (0, i, 0)),
                  pl.BlockSpec((blk, 1), lambda i: (i, 0)),
                  pl.BlockSpec((H, H), lambda i: (0, 0)),
                  pl.BlockSpec((H, H), lambda i: (0, 0)),
                  pl.BlockSpec((1, 1, blk), lambda i: (i, 0, 0)),
                  pl.BlockSpec((H, T), lambda i: (0, 0)),
                  pl.BlockSpec((1, T), lambda i: (0, 0))],
        out_specs=[pl.BlockSpec((blk, H), lambda i: (i, 0)),
                   pl.BlockSpec((G, T), lambda i: (0, 0))],
        out_shape=[jax.ShapeDtypeStruct((NP, H), jnp.float32),
                   jax.ShapeDtypeStruct((G, T), jnp.float32)],
        scratch_shapes=[pltpu.VMEM((G, H), jnp.float32),
                        pltpu.VMEM((G, H), jnp.float32)],
    )(a, inv, wa, wb, batch3d, wout, bout2d)


def kernel(x, edge_index, edge_attr, batch, W0a, W0b, W1a, W1b, W2a, W2b,
           Wout, bout):
    src2d = edge_index[0].reshape(E // CH, CH)
    dst2d = edge_index[1].reshape(E // CH, CH)

    cnt = _count(dst2d)
    x1 = _mlp0(x, W0a, W0b)
    dst2d_g, _ = lax.optimization_barrier((dst2d, cnt[0, 0, 0]))
    agg1 = _scatter_agg(x1, src2d, dst2d_g)
    x2p, inv = _mlp1(agg1, cnt, W1a, W1b)
    agg2 = _scatter_agg(x2p, src2d, dst2d)

    batch_pad = jnp.concatenate(
        [batch, jnp.full((NP - N,), G, jnp.int32)]).reshape(NP // 1280, 1, 1280)
    x3p, p = _mlp2_head(agg2, inv, W2a, W2b, batch_pad,
                        Wout, bout.reshape(1, T))

    node_embeddings = jnp.concatenate([x1, x2p[:N], x3p[:N]], axis=1)
    return (p, node_embeddings)
